# baseline (speedup 1.0000x reference)
import functools

import jax
import jax.numpy as jnp
from jax import lax
from jax.experimental import pallas as pl
from jax.experimental.pallas import tpu as pltpu
from jax.experimental.pallas import tpu_sc as plsc

_B = 16
_C = 1203
_Q = 900
_QS = 896
_QP = 912
_ALPHA = 0.25
_LOG2E = 1.4426950408889634
_LN2 = 0.6931471805599453


def _sc_argmax_body(a_hbm, bout_hbm, cout_hbm, buf, bstage, cstage, sem):
    wid = lax.axis_index("s") * 2 + lax.axis_index("c")
    b = wid // 2
    half = lax.rem(wid, 2)
    row0 = 448 * half
    n_chunks = 56

    lanes = lax.iota(jnp.int32, 16)

    def dma(ch, slot):
        start = row0 + ch * 8
        return pltpu.make_async_copy(
            a_hbm.at[b, pl.ds(start, 8), :],
            buf.at[slot],
            sem.at[slot],
        )

    dma(0, 0).start()

    def body(ch, labvec):
        slot = lax.rem(ch, 2)
        nslot = lax.rem(ch + 1, 2)

        @pl.when(ch + 1 < n_chunks)
        def _():
            dma(ch + 1, nslot).start()

        dma(ch, slot).wait()

        for r in range(8):
            best = jnp.full((16,), -3e38, jnp.float32)
            cvec = jnp.zeros((16,), jnp.int32)
            for j in range(76):
                off = j * 16 if j < 75 else _C - 16
                v = buf[slot, r, pl.ds(off, 16)]
                upd = v > best
                best = jnp.where(upd, v, best)
                cvec = jnp.where(upd, lanes + off, cvec)
            bstage[pl.ds(r * 16, 16)] = best
            cstage[pl.ds(r * 16, 16)] = cvec

        base = (b * _QP + row0 + ch * 8) * 16
        pltpu.sync_copy(bstage, bout_hbm.at[pl.ds(base, 128)])
        pltpu.sync_copy(cstage, cout_hbm.at[pl.ds(base, 128)])
        return 0

    lax.fori_loop(0, n_chunks, body, 0)


def _sc_argmax(a_pred_logits):
    mesh = plsc.VectorSubcoreMesh(core_axis_name="c", subcore_axis_name="s")
    f = functools.partial(
        pl.kernel,
        mesh=mesh,
        out_type=[
            jax.ShapeDtypeStruct((_B * _QP * 16,), jnp.float32),
            jax.ShapeDtypeStruct((_B * _QP * 16,), jnp.int32),
        ],
        scratch_types=[
            pltpu.VMEM((2, 8, _C), jnp.float32),
            pltpu.VMEM((128,), jnp.float32),
            pltpu.VMEM((128,), jnp.int32),
            pltpu.SemaphoreType.DMA((2,)),
        ],
    )(_sc_argmax_body)
    return f(a_pred_logits)


def _tc_body(g_ref, bcand_ref, ccand_ref, atail_ref, src_ref, tgt_ref, out_ref):
    b = pl.program_id(0)
    nb = pl.num_programs(0)

    gv = g_ref[0]
    bt = bcand_ref[0][: _Q]
    ct = ccand_ref[0][: _Q]
    mlane = jnp.max(bt, axis=1, keepdims=True)
    lab_col = jnp.min(jnp.where(bt == mlane, ct, 1 << 20), axis=1, keepdims=True)
    at4 = atail_ref[0]
    s_row = src_ref[b]
    t_row = tgt_ref[b]

    iota_c = lax.broadcasted_iota(jnp.int32, (_Q, _C), 1)
    iota_q0 = lax.broadcasted_iota(jnp.int32, (_Q, _Q), 0)
    iota_col = lax.broadcasted_iota(jnp.int32, (_Q, 1), 0)

    iota_c4 = lax.broadcasted_iota(jnp.int32, (4, _C), 1)
    m4 = jnp.max(at4, axis=1, keepdims=True)
    lab4 = jnp.min(jnp.where(at4 == m4, iota_c4, _C), axis=1, keepdims=True)

    lab_sc = jnp.where(iota_col < _QS, lab_col, 0)
    n_mask = iota_q0 == t_row
    lt_row = jnp.sum(jnp.where(n_mask, lab_sc, 0), axis=0, keepdims=True)
    iota_q4 = lax.broadcasted_iota(jnp.int32, (4, _Q), 0) + _QS
    t_mask4 = iota_q4 == t_row
    lt_row = lt_row + jnp.sum(jnp.where(t_mask4, lab4, 0), axis=0, keepdims=True)

    m_mask = iota_q0 == s_row
    k_col = jnp.sum(jnp.where(m_mask, lt_row, 0), axis=1, keepdims=True)

    y = gv * _LOG2E
    e = jnp.exp2(jnp.minimum(y, -y))
    u = jnp.log2(1.0 + e)
    sp = jnp.maximum(y, 0.0) + u
    z = jnp.minimum(y, 0.0) - u
    s2 = jnp.exp2(z + z)
    l0s = jnp.sum(sp * s2)

    sel = iota_c == k_col
    yk = jnp.sum(jnp.where(sel, y, 0.0), axis=1, keepdims=True)
    ek = jnp.exp2(jnp.minimum(yk, -yk))
    uk = jnp.log2(1.0 + ek)
    sp_p = jnp.maximum(yk, 0.0) + uk
    sp_n = sp_p - yk
    zp = jnp.minimum(yk, 0.0) - uk
    zn = jnp.minimum(-yk, 0.0) - uk
    s2_p = jnp.exp2(zp + zp)
    s2_n = jnp.exp2(zn + zn)
    corr = jnp.sum(_ALPHA * sp_n * s2_n - (1.0 - _ALPHA) * sp_p * s2_p)

    contrib = ((1.0 - _ALPHA) * l0s + corr) * _LN2

    @pl.when(b == 0)
    def _init():
        out_ref[...] = jnp.zeros((1, 1), jnp.float32)

    out_ref[...] += jnp.full((1, 1), contrib, jnp.float32)

    @pl.when(b == nb - 1)
    def _finish():
        out_ref[...] = out_ref[...] / (nb * _Q)


@jax.jit
def _run(g_pred_logits, a_pred_logits, assign_src, assign_tgt):
    B, Q, C = g_pred_logits.shape
    bcand, ccand = _sc_argmax(a_pred_logits)
    bcand = bcand.reshape(B, _QP, 16)
    ccand = ccand.reshape(B, _QP, 16)
    a_tail = lax.slice(a_pred_logits, (0, _QS, 0), (B, Q, C))
    src3 = assign_src.reshape(B, 1, Q)
    tgt3 = assign_tgt.reshape(B, 1, Q)
    out = pl.pallas_call(
        _tc_body,
        grid=(B,),
        in_specs=[
            pl.BlockSpec((1, Q, C), lambda b: (b, 0, 0)),
            pl.BlockSpec((1, _QP, 16), lambda b: (b, 0, 0)),
            pl.BlockSpec((1, _QP, 16), lambda b: (b, 0, 0)),
            pl.BlockSpec((1, 4, C), lambda b: (b, 0, 0)),
            pl.BlockSpec((B, 1, Q), lambda b: (0, 0, 0)),
            pl.BlockSpec((B, 1, Q), lambda b: (0, 0, 0)),
        ],
        out_specs=pl.BlockSpec((1, 1), lambda b: (0, 0)),
        out_shape=jax.ShapeDtypeStruct((1, 1), jnp.float32),
    )(g_pred_logits, bcand, ccand, a_tail, src3, tgt3)
    return out[0, 0]


def kernel(g_pred_logits, a_pred_logits, a_pred_boxes, assign_src, assign_tgt):
    del a_pred_boxes
    return _run(g_pred_logits, a_pred_logits, assign_src, assign_tgt)


# speedup vs baseline: 1.5390x; 1.5390x over previous
import jax
import jax.numpy as jnp
from jax import lax
from jax.experimental import pallas as pl

_C = 1203
_Q = 900
_ALPHA = 0.25
_LOG2E = 1.4426950408889634
_LN2 = 0.6931471805599453


def _body(a_ref, g_ref, src_ref, tgt_ref, out_ref):
    b = pl.program_id(0)
    nb = pl.num_programs(0)

    av = a_ref[0]
    gv = g_ref[0]
    s_row = src_ref[b]
    t_row = tgt_ref[b]

    iota_c = lax.broadcasted_iota(jnp.int32, (_Q, _C), 1)

    m = jnp.max(av, axis=1, keepdims=True)
    lab_col = jnp.min(jnp.where(av == m, iota_c, _C), axis=1, keepdims=True)

    iota_q0 = lax.broadcasted_iota(jnp.int32, (_Q, _Q), 0)
    n_mask = iota_q0 == t_row
    lt_row = jnp.sum(jnp.where(n_mask, lab_col, 0), axis=0, keepdims=True)

    m_mask = iota_q0 == s_row
    k_col = jnp.sum(jnp.where(m_mask, lt_row, 0), axis=1, keepdims=True)

    y = gv * _LOG2E
    e = jnp.exp2(jnp.minimum(y, -y))
    u = jnp.log2(1.0 + e)
    sp = jnp.maximum(y, 0.0) + u
    z = jnp.minimum(y, 0.0) - u
    s2 = jnp.exp2(z + z)
    l0s = jnp.sum(sp * s2)

    sel = iota_c == k_col
    xk = jnp.sum(jnp.where(sel, gv, 0.0), axis=1, keepdims=True)
    yk = xk * _LOG2E
    ek = jnp.exp2(jnp.minimum(yk, -yk))
    uk = jnp.log2(1.0 + ek)
    sp_p = jnp.maximum(yk, 0.0) + uk
    sp_n = sp_p - yk
    zp = jnp.minimum(yk, 0.0) - uk
    zn = jnp.minimum(-yk, 0.0) - uk
    s2_p = jnp.exp2(zp + zp)
    s2_n = jnp.exp2(zn + zn)
    corr = jnp.sum(_ALPHA * sp_n * s2_n - (1.0 - _ALPHA) * sp_p * s2_p)

    contrib = ((1.0 - _ALPHA) * l0s + corr) * _LN2

    @pl.when(b == 0)
    def _init():
        out_ref[...] = jnp.zeros((1, 1), jnp.float32)

    out_ref[...] += jnp.full((1, 1), contrib, jnp.float32)

    @pl.when(b == nb - 1)
    def _finish():
        out_ref[...] = out_ref[...] / (nb * _Q)


@jax.jit
def _run(g_pred_logits, a_pred_logits, assign_src, assign_tgt):
    B, Q, C = g_pred_logits.shape
    src3 = assign_src.reshape(B, 1, Q)
    tgt3 = assign_tgt.reshape(B, 1, Q)
    out = pl.pallas_call(
        _body,
        grid=(B,),
        in_specs=[
            pl.BlockSpec((1, Q, C), lambda b: (b, 0, 0)),
            pl.BlockSpec((1, Q, C), lambda b: (b, 0, 0)),
            pl.BlockSpec((B, 1, Q), lambda b: (0, 0, 0)),
            pl.BlockSpec((B, 1, Q), lambda b: (0, 0, 0)),
        ],
        out_specs=pl.BlockSpec((1, 1), lambda b: (0, 0)),
        out_shape=jax.ShapeDtypeStruct((1, 1), jnp.float32),
    )(a_pred_logits, g_pred_logits, src3, tgt3)
    return out[0, 0]


def kernel(g_pred_logits, a_pred_logits, a_pred_boxes, assign_src, assign_tgt):
    del a_pred_boxes
    return _run(g_pred_logits, a_pred_logits, assign_src, assign_tgt)
